# baseline (speedup 1.0000x reference)
import functools

import jax
import jax.numpy as jnp
from jax import lax
from jax.experimental import pallas as pl
from jax.experimental.pallas import tpu as pltpu
from jax.experimental.pallas import tpu_sc as plsc

_NC = 2
_NS = 16
_NW = _NC * _NS
_L = 16

_POS_THR = 0.5
_NEG_THR = 0.4
_BIG = 2e9


def _build_sc_call(n_pad, g_pad, per_w, k_unroll, g_unroll=1):
    n_chunks = per_w // (k_unroll * _L)

    def body(anc_hbm, gt_hbm, inds_hbm, mo_hbm, labs_hbm,
             ax0_v, ay0_v, ax1_v, ay1_v,
             gx0_v, gy0_v, gx1_v, gy1_v, glab_v, garea_v, compact_v,
             oinds_v, omo_v, olabs_v):
        wid = lax.axis_index("s") * _NC + lax.axis_index("c")
        base = wid * per_w

        pltpu.sync_copy(anc_hbm.at[pl.ds(0 * n_pad + base, per_w)], ax0_v)
        pltpu.sync_copy(anc_hbm.at[pl.ds(1 * n_pad + base, per_w)], ay0_v)
        pltpu.sync_copy(anc_hbm.at[pl.ds(2 * n_pad + base, per_w)], ax1_v)
        pltpu.sync_copy(anc_hbm.at[pl.ds(3 * n_pad + base, per_w)], ay1_v)
        pltpu.sync_copy(gt_hbm.at[pl.ds(0 * g_pad, g_pad)], gx0_v)
        pltpu.sync_copy(gt_hbm.at[pl.ds(1 * g_pad, g_pad)], gy0_v)
        pltpu.sync_copy(gt_hbm.at[pl.ds(2 * g_pad, g_pad)], gx1_v)
        pltpu.sync_copy(gt_hbm.at[pl.ds(3 * g_pad, g_pad)], gy1_v)
        pltpu.sync_copy(gt_hbm.at[pl.ds(4 * g_pad, g_pad)], glab_v)

        big_v = jnp.full((_L,), _BIG, jnp.float32)
        neg1f_v = jnp.full((_L,), -1.0, jnp.float32)
        zero_v = jnp.zeros((_L,), jnp.float32)
        eps_v = jnp.full((_L,), 1e-6, jnp.float32)
        pos_v = jnp.full((_L,), _POS_THR, jnp.float32)
        negthr_v = jnp.full((_L,), _NEG_THR, jnp.float32)
        one_iv = jnp.full((_L,), 1, jnp.int32)
        neg1_iv = jnp.full((_L,), -1, jnp.int32)
        zero_iv = jnp.zeros((_L,), jnp.int32)

        off_v = zero_iv - one_iv
        for j in range(g_pad // _L):
            sl = pl.ds(j * _L, _L)
            valid = glab_v[sl] != neg1f_v
            gx0 = jnp.where(valid, gx0_v[sl], big_v)
            gy0 = jnp.where(valid, gy0_v[sl], big_v)
            gx1 = jnp.where(valid, gx1_v[sl], big_v)
            gy1 = jnp.where(valid, gy1_v[sl], big_v)
            gx0_v[sl] = gx0
            gy0_v[sl] = gy0
            gx1_v[sl] = gx1
            gy1_v[sl] = gy1
            garea_v[sl] = (gx1 - gx0) * (gy1 - gy0)
            vi = jnp.where(valid, one_iv, zero_iv)
            cum = jnp.cumsum(vi)
            compact_v[sl] = cum + off_v
            off_v = plsc.load_gather(
                compact_v, [jnp.full((_L,), j * _L + _L - 1, jnp.int32)])
        for oc in range(n_chunks):
            cbase = oc * k_unroll * _L
            anchors = []
            for k in range(k_unroll):
                sl = pl.ds(cbase + k * _L, _L)
                x0 = ax0_v[sl]
                y0 = ay0_v[sl]
                x1 = ax1_v[sl]
                y1 = ay1_v[sl]
                anchors.append((x0, y0, x1, y1, (x1 - x0) * (y1 - y0)))

            def step(jj, carry, anchors=anchors):
                bious, bargs = carry
                for u in range(g_unroll):
                    j = jj * g_unroll + u if g_unroll > 1 else jj
                    idx = jnp.full((_L,), j, dtype=jnp.int32)
                    gx0 = plsc.load_gather(gx0_v, [idx])
                    gy0 = plsc.load_gather(gy0_v, [idx])
                    gx1 = plsc.load_gather(gx1_v, [idx])
                    gy1 = plsc.load_gather(gy1_v, [idx])
                    ga = plsc.load_gather(garea_v, [idx])
                    nb = []
                    na = []
                    for k in range(k_unroll):
                        x0, y0, x1, y1, aa = anchors[k]
                        w = jnp.maximum(jnp.minimum(x1, gx1) - jnp.maximum(x0, gx0), zero_v)
                        h = jnp.maximum(jnp.minimum(y1, gy1) - jnp.maximum(y0, gy0), zero_v)
                        inter = w * h
                        den = ((aa + ga) - inter) + eps_v
                        iou = inter / den
                        upd = iou > bious[k]
                        nb.append(jnp.where(upd, iou, bious[k]))
                        na.append(jnp.where(upd, idx, bargs[k]))
                    bious, bargs = tuple(nb), tuple(na)
                return (bious, bargs)

            init = (tuple(neg1f_v for _ in range(k_unroll)),
                    tuple(zero_iv for _ in range(k_unroll)))
            bious, bargs = lax.fori_loop(0, g_pad // g_unroll, step, init)

            for k in range(k_unroll):
                sl = pl.ds(cbase + k * _L, _L)
                biou = bious[k]
                barg = bargs[k]
                pos = biou > pos_v
                neg = biou < negthr_v
                cid = plsc.load_gather(compact_v, [barg])
                labi = plsc.load_gather(glab_v, [barg]).astype(jnp.int32)
                oinds_v[sl] = jnp.where(pos, cid + one_iv, jnp.where(neg, zero_iv, neg1_iv))
                omo_v[sl] = biou
                olabs_v[sl] = jnp.where(pos, labi, neg1_iv)

        pltpu.sync_copy(oinds_v, inds_hbm.at[pl.ds(base, per_w)])
        pltpu.sync_copy(omo_v, mo_hbm.at[pl.ds(base, per_w)])
        pltpu.sync_copy(olabs_v, labs_hbm.at[pl.ds(base, per_w)])

    return pl.kernel(
        body,
        out_type=(
            jax.ShapeDtypeStruct((n_pad,), jnp.int32),
            jax.ShapeDtypeStruct((n_pad,), jnp.float32),
            jax.ShapeDtypeStruct((n_pad,), jnp.int32),
        ),
        mesh=plsc.VectorSubcoreMesh(
            core_axis_name="c", subcore_axis_name="s",
            num_cores=_NC, num_subcores=_NS),
        compiler_params=pltpu.CompilerParams(needs_layout_passes=False),
        scratch_types=[
            pltpu.VMEM((per_w,), jnp.float32),
            pltpu.VMEM((per_w,), jnp.float32),
            pltpu.VMEM((per_w,), jnp.float32),
            pltpu.VMEM((per_w,), jnp.float32),
            pltpu.VMEM((g_pad,), jnp.float32),
            pltpu.VMEM((g_pad,), jnp.float32),
            pltpu.VMEM((g_pad,), jnp.float32),
            pltpu.VMEM((g_pad,), jnp.float32),
            pltpu.VMEM((g_pad,), jnp.float32),
            pltpu.VMEM((g_pad,), jnp.float32),
            pltpu.VMEM((g_pad,), jnp.int32),
            pltpu.VMEM((per_w,), jnp.int32),
            pltpu.VMEM((per_w,), jnp.float32),
            pltpu.VMEM((per_w,), jnp.int32),
        ],
    )


_TC_R = 32


def _build_tc_call(nbt, g_pad):
    def body(gt_smem, anc_ref, inds_ref, mo_ref, labs_ref, coords_s, cid_s):
        bigf = jnp.float32(_BIG)

        @pl.when(pl.program_id(0) == 0)
        def _prologue():
            def pro(j, cnt):
                lab = gt_smem[4 * g_pad + j]
                valid = lab != -1.0
                gx0 = jnp.where(valid, gt_smem[0 * g_pad + j], bigf)
                gy0 = jnp.where(valid, gt_smem[1 * g_pad + j], bigf)
                gx1 = jnp.where(valid, gt_smem[2 * g_pad + j], bigf)
                gy1 = jnp.where(valid, gt_smem[3 * g_pad + j], bigf)
                coords_s[0, j] = gx0
                coords_s[1, j] = gy0
                coords_s[2, j] = gx1
                coords_s[3, j] = gy1
                coords_s[4, j] = (gx1 - gx0) * (gy1 - gy0)
                cnt = cnt + valid.astype(jnp.int32)
                cid_s[j] = cnt - 1
                return cnt
            lax.fori_loop(0, g_pad, pro, jnp.int32(0))

        x0 = anc_ref[0]
        y0 = anc_ref[1]
        x1 = anc_ref[2]
        y1 = anc_ref[3]
        aa = (x1 - x0) * (y1 - y0)

        def step(j, carry):
            biou, bcid, blab = carry
            gx0 = coords_s[0, j]
            gy0 = coords_s[1, j]
            gx1 = coords_s[2, j]
            gy1 = coords_s[3, j]
            ga = coords_s[4, j]
            w = jnp.maximum(jnp.minimum(x1, gx1) - jnp.maximum(x0, gx0), 0.0)
            h = jnp.maximum(jnp.minimum(y1, gy1) - jnp.maximum(y0, gy0), 0.0)
            inter = w * h
            den = ((aa + ga) - inter) + jnp.float32(1e-6)
            iou = inter / den
            upd = iou > biou
            biou = jnp.where(upd, iou, biou)
            bcid = jnp.where(upd, cid_s[j], bcid)
            blab = jnp.where(upd, gt_smem[4 * g_pad + j], blab)
            return (biou, bcid, blab)

        shape = x0.shape
        init = (jnp.full(shape, -1.0, jnp.float32),
                jnp.zeros(shape, jnp.int32),
                jnp.full(shape, -1.0, jnp.float32))
        biou, bcid, blab = lax.fori_loop(0, g_pad, step, init)
        pos = biou > _POS_THR
        neg = biou < _NEG_THR
        inds_ref[...] = jnp.where(pos, bcid + 1, jnp.where(neg, 0, -1))
        mo_ref[...] = biou
        labs_ref[...] = jnp.where(pos, blab.astype(jnp.int32), -1)

    return pl.pallas_call(
        body,
        grid=(nbt,),
        in_specs=[
            pl.BlockSpec(memory_space=pltpu.SMEM),
            pl.BlockSpec((4, _TC_R, 128), lambda i: (0, i, 0)),
        ],
        out_specs=[
            pl.BlockSpec((_TC_R, 128), lambda i: (i, 0)),
            pl.BlockSpec((_TC_R, 128), lambda i: (i, 0)),
            pl.BlockSpec((_TC_R, 128), lambda i: (i, 0)),
        ],
        out_shape=[
            jax.ShapeDtypeStruct((nbt * _TC_R, 128), jnp.int32),
            jax.ShapeDtypeStruct((nbt * _TC_R, 128), jnp.float32),
            jax.ShapeDtypeStruct((nbt * _TC_R, 128), jnp.int32),
        ],
        scratch_shapes=[
            pltpu.SMEM((5, g_pad), jnp.float32),
            pltpu.SMEM((g_pad,), jnp.int32),
        ],
    )


_SC_FRAC = 0.615


def kernel(bboxes, targets, num_level_bboxes):
    n = bboxes.shape[0]
    g = targets.shape[0]
    g_pad = -(-g // _L) * _L

    tgt = jnp.pad(targets, ((0, g_pad - g), (0, 0)),
                  constant_values=-1.0)
    gt_flat = tgt.T.reshape(-1)
    anc_t = bboxes.T

    sc_quantum = _NW * _L * 4
    n_sc = int(n * _SC_FRAC) // sc_quantum * sc_quantum
    n_tc = n - n_sc

    outs = []
    if n_sc > 0:
        per_w = n_sc // _NW
        sc_call = _build_sc_call(n_sc, g_pad, per_w, k_unroll=4, g_unroll=1)
        sc_out = sc_call(anc_t[:, :n_sc].reshape(-1), gt_flat)
        outs.append(sc_out)
    if n_tc > 0:
        nbt = -(-n_tc // (_TC_R * 128))
        n_tc_pad = nbt * _TC_R * 128
        anc_tc = jnp.pad(anc_t[:, n_sc:], ((0, 0), (0, n_tc_pad - n_tc)))
        tc_call = _build_tc_call(nbt, g_pad)
        tc_out = tc_call(gt_flat, anc_tc.reshape(4, nbt * _TC_R, 128))
        outs.append(tuple(o.reshape(-1)[:n_tc] for o in tc_out))

    if len(outs) == 2:
        inds, mo, labs = (jnp.concatenate([a, b]) for a, b in zip(outs[0], outs[1]))
    else:
        inds, mo, labs = outs[0]
    return (inds.astype(jnp.int64),
            mo,
            labs.astype(jnp.int64))


# speedup vs baseline: 1.1811x; 1.1811x over previous
import functools

import jax
import jax.numpy as jnp
from jax import lax
from jax.experimental import pallas as pl
from jax.experimental.pallas import tpu as pltpu
from jax.experimental.pallas import tpu_sc as plsc

_NC = 2
_NS = 16
_NW = _NC * _NS
_L = 16

_POS_THR = 0.5
_NEG_THR = 0.4
_BIG = 2e9


def _build_sc_call(n_pad, g_pad, per_w, k_unroll, g_unroll=1):
    n_chunks = per_w // (k_unroll * _L)

    def body(anc_hbm, gt_hbm, inds_hbm, mo_hbm, labs_hbm,
             ax0_v, ay0_v, ax1_v, ay1_v,
             gx0_v, gy0_v, gx1_v, gy1_v, glab_v, garea_v, compact_v,
             oinds_v, omo_v, olabs_v):
        wid = lax.axis_index("s") * _NC + lax.axis_index("c")
        base = wid * per_w

        pltpu.sync_copy(anc_hbm.at[pl.ds(0 * n_pad + base, per_w)], ax0_v)
        pltpu.sync_copy(anc_hbm.at[pl.ds(1 * n_pad + base, per_w)], ay0_v)
        pltpu.sync_copy(anc_hbm.at[pl.ds(2 * n_pad + base, per_w)], ax1_v)
        pltpu.sync_copy(anc_hbm.at[pl.ds(3 * n_pad + base, per_w)], ay1_v)
        pltpu.sync_copy(gt_hbm.at[pl.ds(0 * g_pad, g_pad)], gx0_v)
        pltpu.sync_copy(gt_hbm.at[pl.ds(1 * g_pad, g_pad)], gy0_v)
        pltpu.sync_copy(gt_hbm.at[pl.ds(2 * g_pad, g_pad)], gx1_v)
        pltpu.sync_copy(gt_hbm.at[pl.ds(3 * g_pad, g_pad)], gy1_v)
        pltpu.sync_copy(gt_hbm.at[pl.ds(4 * g_pad, g_pad)], glab_v)

        big_v = jnp.full((_L,), _BIG, jnp.float32)
        neg1f_v = jnp.full((_L,), -1.0, jnp.float32)
        zero_v = jnp.zeros((_L,), jnp.float32)
        eps_v = jnp.full((_L,), 1e-6, jnp.float32)
        pos_v = jnp.full((_L,), _POS_THR, jnp.float32)
        negthr_v = jnp.full((_L,), _NEG_THR, jnp.float32)
        one_iv = jnp.full((_L,), 1, jnp.int32)
        neg1_iv = jnp.full((_L,), -1, jnp.int32)
        zero_iv = jnp.zeros((_L,), jnp.int32)

        off_v = zero_iv - one_iv
        for j in range(g_pad // _L):
            sl = pl.ds(j * _L, _L)
            valid = glab_v[sl] != neg1f_v
            gx0 = jnp.where(valid, gx0_v[sl], big_v)
            gy0 = jnp.where(valid, gy0_v[sl], big_v)
            gx1 = jnp.where(valid, gx1_v[sl], big_v)
            gy1 = jnp.where(valid, gy1_v[sl], big_v)
            gx0_v[sl] = gx0
            gy0_v[sl] = gy0
            gx1_v[sl] = gx1
            gy1_v[sl] = gy1
            garea_v[sl] = (gx1 - gx0) * (gy1 - gy0)
            vi = jnp.where(valid, one_iv, zero_iv)
            cum = jnp.cumsum(vi)
            compact_v[sl] = cum + off_v
            off_v = plsc.load_gather(
                compact_v, [jnp.full((_L,), j * _L + _L - 1, jnp.int32)])
        for oc in range(n_chunks):
            cbase = oc * k_unroll * _L
            anchors = []
            for k in range(k_unroll):
                sl = pl.ds(cbase + k * _L, _L)
                x0 = ax0_v[sl]
                y0 = ay0_v[sl]
                x1 = ax1_v[sl]
                y1 = ay1_v[sl]
                anchors.append((x0, y0, x1, y1, (x1 - x0) * (y1 - y0)))

            def step(jj, carry, anchors=anchors):
                bious, bargs = carry
                for u in range(g_unroll):
                    j = jj * g_unroll + u if g_unroll > 1 else jj
                    idx = jnp.full((_L,), j, dtype=jnp.int32)
                    gx0 = plsc.load_gather(gx0_v, [idx])
                    gy0 = plsc.load_gather(gy0_v, [idx])
                    gx1 = plsc.load_gather(gx1_v, [idx])
                    gy1 = plsc.load_gather(gy1_v, [idx])
                    ga = plsc.load_gather(garea_v, [idx])
                    nb = []
                    na = []
                    for k in range(k_unroll):
                        x0, y0, x1, y1, aa = anchors[k]
                        w = jnp.maximum(jnp.minimum(x1, gx1) - jnp.maximum(x0, gx0), zero_v)
                        h = jnp.maximum(jnp.minimum(y1, gy1) - jnp.maximum(y0, gy0), zero_v)
                        inter = w * h
                        den = ((aa + ga) - inter) + eps_v
                        iou = inter / den
                        upd = iou > bious[k]
                        nb.append(jnp.where(upd, iou, bious[k]))
                        na.append(jnp.where(upd, idx, bargs[k]))
                    bious, bargs = tuple(nb), tuple(na)
                return (bious, bargs)

            init = (tuple(neg1f_v for _ in range(k_unroll)),
                    tuple(zero_iv for _ in range(k_unroll)))
            bious, bargs = lax.fori_loop(0, g_pad // g_unroll, step, init)

            for k in range(k_unroll):
                sl = pl.ds(cbase + k * _L, _L)
                biou = bious[k]
                barg = bargs[k]
                pos = biou > pos_v
                neg = biou < negthr_v
                cid = plsc.load_gather(compact_v, [barg])
                labi = plsc.load_gather(glab_v, [barg]).astype(jnp.int32)
                oinds_v[sl] = jnp.where(pos, cid + one_iv, jnp.where(neg, zero_iv, neg1_iv))
                omo_v[sl] = biou
                olabs_v[sl] = jnp.where(pos, labi, neg1_iv)

        pltpu.sync_copy(oinds_v, inds_hbm.at[pl.ds(base, per_w)])
        pltpu.sync_copy(omo_v, mo_hbm.at[pl.ds(base, per_w)])
        pltpu.sync_copy(olabs_v, labs_hbm.at[pl.ds(base, per_w)])

    return pl.kernel(
        body,
        cost_estimate=pl.CostEstimate(
            flops=22 * g_pad * n_pad, transcendentals=0,
            bytes_accessed=28 * n_pad + 20 * g_pad),
        out_type=(
            jax.ShapeDtypeStruct((n_pad,), jnp.int32),
            jax.ShapeDtypeStruct((n_pad,), jnp.float32),
            jax.ShapeDtypeStruct((n_pad,), jnp.int32),
        ),
        mesh=plsc.VectorSubcoreMesh(
            core_axis_name="c", subcore_axis_name="s",
            num_cores=_NC, num_subcores=_NS),
        compiler_params=pltpu.CompilerParams(needs_layout_passes=False),
        scratch_types=[
            pltpu.VMEM((per_w,), jnp.float32),
            pltpu.VMEM((per_w,), jnp.float32),
            pltpu.VMEM((per_w,), jnp.float32),
            pltpu.VMEM((per_w,), jnp.float32),
            pltpu.VMEM((g_pad,), jnp.float32),
            pltpu.VMEM((g_pad,), jnp.float32),
            pltpu.VMEM((g_pad,), jnp.float32),
            pltpu.VMEM((g_pad,), jnp.float32),
            pltpu.VMEM((g_pad,), jnp.float32),
            pltpu.VMEM((g_pad,), jnp.float32),
            pltpu.VMEM((g_pad,), jnp.int32),
            pltpu.VMEM((per_w,), jnp.int32),
            pltpu.VMEM((per_w,), jnp.float32),
            pltpu.VMEM((per_w,), jnp.int32),
        ],
    )


_TC_R = 48


def _build_tc_call(nbt, g_pad):
    def body(gt_smem, anc_ref, inds_ref, mo_ref, labs_ref, coords_s, cid_s):
        bigf = jnp.float32(_BIG)

        @pl.when(pl.program_id(0) == 0)
        def _prologue():
            def pro(j, cnt):
                lab = gt_smem[4 * g_pad + j]
                valid = lab != -1.0
                gx0 = jnp.where(valid, gt_smem[0 * g_pad + j], bigf)
                gy0 = jnp.where(valid, gt_smem[1 * g_pad + j], bigf)
                gx1 = jnp.where(valid, gt_smem[2 * g_pad + j], bigf)
                gy1 = jnp.where(valid, gt_smem[3 * g_pad + j], bigf)
                coords_s[0, j] = gx0
                coords_s[1, j] = gy0
                coords_s[2, j] = gx1
                coords_s[3, j] = gy1
                coords_s[4, j] = (gx1 - gx0) * (gy1 - gy0)
                cnt = cnt + valid.astype(jnp.int32)
                cid_s[j] = cnt - 1
                return cnt
            lax.fori_loop(0, g_pad, pro, jnp.int32(0))

        x0 = anc_ref[0]
        y0 = anc_ref[1]
        x1 = anc_ref[2]
        y1 = anc_ref[3]
        aa = (x1 - x0) * (y1 - y0)

        def step(j, carry):
            biou, bcid, blab = carry
            gx0 = coords_s[0, j]
            gy0 = coords_s[1, j]
            gx1 = coords_s[2, j]
            gy1 = coords_s[3, j]
            ga = coords_s[4, j]
            w = jnp.maximum(jnp.minimum(x1, gx1) - jnp.maximum(x0, gx0), 0.0)
            h = jnp.maximum(jnp.minimum(y1, gy1) - jnp.maximum(y0, gy0), 0.0)
            inter = w * h
            den = ((aa + ga) - inter) + jnp.float32(1e-6)
            iou = inter / den
            upd = iou > biou
            biou = jnp.where(upd, iou, biou)
            bcid = jnp.where(upd, cid_s[j], bcid)
            blab = jnp.where(upd, gt_smem[4 * g_pad + j], blab)
            return (biou, bcid, blab)

        shape = x0.shape
        init = (jnp.full(shape, -1.0, jnp.float32),
                jnp.zeros(shape, jnp.int32),
                jnp.full(shape, -1.0, jnp.float32))
        biou, bcid, blab = lax.fori_loop(0, g_pad, step, init)
        pos = biou > _POS_THR
        neg = biou < _NEG_THR
        inds_ref[...] = jnp.where(pos, bcid + 1, jnp.where(neg, 0, -1))
        mo_ref[...] = biou
        labs_ref[...] = jnp.where(pos, blab.astype(jnp.int32), -1)

    return pl.pallas_call(
        body,
        grid=(nbt,),
        cost_estimate=pl.CostEstimate(
            flops=22 * g_pad * nbt * _TC_R * 128, transcendentals=0,
            bytes_accessed=28 * nbt * _TC_R * 128 + 20 * g_pad),
        in_specs=[
            pl.BlockSpec(memory_space=pltpu.SMEM),
            pl.BlockSpec((4, _TC_R, 128), lambda i: (0, i, 0)),
        ],
        out_specs=[
            pl.BlockSpec((_TC_R, 128), lambda i: (i, 0)),
            pl.BlockSpec((_TC_R, 128), lambda i: (i, 0)),
            pl.BlockSpec((_TC_R, 128), lambda i: (i, 0)),
        ],
        out_shape=[
            jax.ShapeDtypeStruct((nbt * _TC_R, 128), jnp.int32),
            jax.ShapeDtypeStruct((nbt * _TC_R, 128), jnp.float32),
            jax.ShapeDtypeStruct((nbt * _TC_R, 128), jnp.int32),
        ],
        scratch_shapes=[
            pltpu.SMEM((5, g_pad), jnp.float32),
            pltpu.SMEM((g_pad,), jnp.int32),
        ],
    )


_SC_FRAC = 0.42


def kernel(bboxes, targets, num_level_bboxes):
    n = bboxes.shape[0]
    g = targets.shape[0]
    g_pad = -(-g // _L) * _L

    tgt = jnp.pad(targets, ((0, g_pad - g), (0, 0)),
                  constant_values=-1.0)
    gt_flat = tgt.T.reshape(-1)
    anc_t = bboxes.T

    sc_quantum = _NW * _L * 4
    n_sc = int(n * _SC_FRAC) // sc_quantum * sc_quantum
    n_tc = n - n_sc

    outs = []
    if n_sc > 0:
        per_w = n_sc // _NW
        sc_call = _build_sc_call(n_sc, g_pad, per_w, k_unroll=4, g_unroll=1)
        sc_out = sc_call(anc_t[:, :n_sc].reshape(-1), gt_flat)
        outs.append(sc_out)
    if n_tc > 0:
        nbt = -(-n_tc // (_TC_R * 128))
        n_tc_pad = nbt * _TC_R * 128
        anc_tc = jnp.pad(anc_t[:, n_sc:], ((0, 0), (0, n_tc_pad - n_tc)))
        tc_call = _build_tc_call(nbt, g_pad)
        tc_out = tc_call(gt_flat, anc_tc.reshape(4, nbt * _TC_R, 128))
        outs.append(tuple(o.reshape(-1)[:n_tc] for o in tc_out))

    if len(outs) == 2:
        inds, mo, labs = (jnp.concatenate([a, b]) for a, b in zip(outs[0], outs[1]))
    else:
        inds, mo, labs = outs[0]
    return (inds.astype(jnp.int64),
            mo,
            labs.astype(jnp.int64))


# speedup vs baseline: 1.1949x; 1.0117x over previous
import jax
import jax.numpy as jnp
from jax import lax
from jax.experimental import pallas as pl
from jax.experimental.pallas import tpu as pltpu
from jax.experimental.pallas import tpu_sc as plsc

_NC = 2
_NS = 16
_NW = _NC * _NS
_L = 16

_POS_THR = 0.5
_NEG_THR = 0.4
_BIG = 2e9


def _build_sc_call(n_pad, g_pad, per_w, k_unroll, g_unroll=1):
    n_chunks = per_w // (k_unroll * _L)

    def body(anc_hbm, gt_hbm, inds_hbm, mo_hbm, labs_hbm,
             ax0_v, ay0_v, ax1_v, ay1_v,
             gx0_v, gy0_v, gx1_v, gy1_v, glab_v, garea_v, compact_v,
             oinds_v, omo_v, olabs_v):
        wid = lax.axis_index("s") * _NC + lax.axis_index("c")
        base = wid * per_w

        pltpu.sync_copy(anc_hbm.at[pl.ds(0 * n_pad + base, per_w)], ax0_v)
        pltpu.sync_copy(anc_hbm.at[pl.ds(1 * n_pad + base, per_w)], ay0_v)
        pltpu.sync_copy(anc_hbm.at[pl.ds(2 * n_pad + base, per_w)], ax1_v)
        pltpu.sync_copy(anc_hbm.at[pl.ds(3 * n_pad + base, per_w)], ay1_v)
        pltpu.sync_copy(gt_hbm.at[pl.ds(0 * g_pad, g_pad)], gx0_v)
        pltpu.sync_copy(gt_hbm.at[pl.ds(1 * g_pad, g_pad)], gy0_v)
        pltpu.sync_copy(gt_hbm.at[pl.ds(2 * g_pad, g_pad)], gx1_v)
        pltpu.sync_copy(gt_hbm.at[pl.ds(3 * g_pad, g_pad)], gy1_v)
        pltpu.sync_copy(gt_hbm.at[pl.ds(4 * g_pad, g_pad)], glab_v)

        big_v = jnp.full((_L,), _BIG, jnp.float32)
        neg1f_v = jnp.full((_L,), -1.0, jnp.float32)
        zero_v = jnp.zeros((_L,), jnp.float32)
        eps_v = jnp.full((_L,), 1e-6, jnp.float32)
        pos_v = jnp.full((_L,), _POS_THR, jnp.float32)
        negthr_v = jnp.full((_L,), _NEG_THR, jnp.float32)
        one_iv = jnp.full((_L,), 1, jnp.int32)
        neg1_iv = jnp.full((_L,), -1, jnp.int32)
        zero_iv = jnp.zeros((_L,), jnp.int32)

        off_v = zero_iv - one_iv
        for j in range(g_pad // _L):
            sl = pl.ds(j * _L, _L)
            valid = glab_v[sl] != neg1f_v
            gx0 = jnp.where(valid, gx0_v[sl], big_v)
            gy0 = jnp.where(valid, gy0_v[sl], big_v)
            gx1 = jnp.where(valid, gx1_v[sl], big_v)
            gy1 = jnp.where(valid, gy1_v[sl], big_v)
            gx0_v[sl] = gx0
            gy0_v[sl] = gy0
            gx1_v[sl] = gx1
            gy1_v[sl] = gy1
            garea_v[sl] = (gx1 - gx0) * (gy1 - gy0)
            vi = jnp.where(valid, one_iv, zero_iv)
            cum = jnp.cumsum(vi)
            compact_v[sl] = cum + off_v
            off_v = plsc.load_gather(
                compact_v, [jnp.full((_L,), j * _L + _L - 1, jnp.int32)])
        for oc in range(n_chunks):
            cbase = oc * k_unroll * _L
            anchors = []
            for k in range(k_unroll):
                sl = pl.ds(cbase + k * _L, _L)
                x0 = ax0_v[sl]
                y0 = ay0_v[sl]
                x1 = ax1_v[sl]
                y1 = ay1_v[sl]
                anchors.append((x0, y0, x1, y1, (x1 - x0) * (y1 - y0)))

            def step(jj, carry, anchors=anchors):
                bious, bargs = carry
                for u in range(g_unroll):
                    j = jj * g_unroll + u if g_unroll > 1 else jj
                    idx = jnp.full((_L,), j, dtype=jnp.int32)
                    gx0 = plsc.load_gather(gx0_v, [idx])
                    gy0 = plsc.load_gather(gy0_v, [idx])
                    gx1 = plsc.load_gather(gx1_v, [idx])
                    gy1 = plsc.load_gather(gy1_v, [idx])
                    ga = plsc.load_gather(garea_v, [idx])
                    nb = []
                    na = []
                    for k in range(k_unroll):
                        x0, y0, x1, y1, aa = anchors[k]
                        w = jnp.maximum(jnp.minimum(x1, gx1) - jnp.maximum(x0, gx0), zero_v)
                        h = jnp.maximum(jnp.minimum(y1, gy1) - jnp.maximum(y0, gy0), zero_v)
                        inter = w * h
                        den = ((aa + ga) - inter) + eps_v
                        iou = inter / den
                        upd = iou > bious[k]
                        nb.append(jnp.where(upd, iou, bious[k]))
                        na.append(jnp.where(upd, idx, bargs[k]))
                    bious, bargs = tuple(nb), tuple(na)
                return (bious, bargs)

            init = (tuple(neg1f_v for _ in range(k_unroll)),
                    tuple(zero_iv for _ in range(k_unroll)))
            bious, bargs = lax.fori_loop(0, g_pad // g_unroll, step, init)

            for k in range(k_unroll):
                sl = pl.ds(cbase + k * _L, _L)
                biou = bious[k]
                barg = bargs[k]
                pos = biou > pos_v
                neg = biou < negthr_v
                cid = plsc.load_gather(compact_v, [barg])
                labi = plsc.load_gather(glab_v, [barg]).astype(jnp.int32)
                oinds_v[sl] = jnp.where(pos, cid + one_iv, jnp.where(neg, zero_iv, neg1_iv))
                omo_v[sl] = biou
                olabs_v[sl] = jnp.where(pos, labi, neg1_iv)

        pltpu.sync_copy(oinds_v, inds_hbm.at[pl.ds(base, per_w)])
        pltpu.sync_copy(omo_v, mo_hbm.at[pl.ds(base, per_w)])
        pltpu.sync_copy(olabs_v, labs_hbm.at[pl.ds(base, per_w)])

    return pl.kernel(
        body,
        out_type=(
            jax.ShapeDtypeStruct((n_pad,), jnp.int32),
            jax.ShapeDtypeStruct((n_pad,), jnp.float32),
            jax.ShapeDtypeStruct((n_pad,), jnp.int32),
        ),
        mesh=plsc.VectorSubcoreMesh(
            core_axis_name="c", subcore_axis_name="s",
            num_cores=_NC, num_subcores=_NS),
        compiler_params=pltpu.CompilerParams(needs_layout_passes=False),
        scratch_types=[
            pltpu.VMEM((per_w,), jnp.float32),
            pltpu.VMEM((per_w,), jnp.float32),
            pltpu.VMEM((per_w,), jnp.float32),
            pltpu.VMEM((per_w,), jnp.float32),
            pltpu.VMEM((g_pad,), jnp.float32),
            pltpu.VMEM((g_pad,), jnp.float32),
            pltpu.VMEM((g_pad,), jnp.float32),
            pltpu.VMEM((g_pad,), jnp.float32),
            pltpu.VMEM((g_pad,), jnp.float32),
            pltpu.VMEM((g_pad,), jnp.float32),
            pltpu.VMEM((g_pad,), jnp.int32),
            pltpu.VMEM((per_w,), jnp.int32),
            pltpu.VMEM((per_w,), jnp.float32),
            pltpu.VMEM((per_w,), jnp.int32),
        ],
    )


_TC_R = 48


def _build_tc_call(nbt, g_pad):
    def body(gt_smem, anc_ref, inds_ref, mo_ref, labs_ref, coords_s, cid_s):
        bigf = jnp.float32(_BIG)

        @pl.when(pl.program_id(0) == 0)
        def _prologue():
            def pro(j, cnt):
                lab = gt_smem[4 * g_pad + j]
                valid = lab != -1.0
                gx0 = jnp.where(valid, gt_smem[0 * g_pad + j], bigf)
                gy0 = jnp.where(valid, gt_smem[1 * g_pad + j], bigf)
                gx1 = jnp.where(valid, gt_smem[2 * g_pad + j], bigf)
                gy1 = jnp.where(valid, gt_smem[3 * g_pad + j], bigf)
                coords_s[0, j] = gx0
                coords_s[1, j] = gy0
                coords_s[2, j] = gx1
                coords_s[3, j] = gy1
                coords_s[4, j] = (gx1 - gx0) * (gy1 - gy0)
                cnt = cnt + valid.astype(jnp.int32)
                cid_s[j] = cnt - 1
                return cnt
            lax.fori_loop(0, g_pad, pro, jnp.int32(0))

        x0 = anc_ref[0]
        y0 = anc_ref[1]
        x1 = anc_ref[2]
        y1 = anc_ref[3]
        aa = (x1 - x0) * (y1 - y0)

        def step(j, carry):
            biou, bcid, blab = carry
            gx0 = coords_s[0, j]
            gy0 = coords_s[1, j]
            gx1 = coords_s[2, j]
            gy1 = coords_s[3, j]
            ga = coords_s[4, j]
            w = jnp.maximum(jnp.minimum(x1, gx1) - jnp.maximum(x0, gx0), 0.0)
            h = jnp.maximum(jnp.minimum(y1, gy1) - jnp.maximum(y0, gy0), 0.0)
            inter = w * h
            den = ((aa + ga) - inter) + jnp.float32(1e-6)
            iou = inter / den
            upd = iou > biou
            biou = jnp.where(upd, iou, biou)
            bcid = jnp.where(upd, cid_s[j], bcid)
            blab = jnp.where(upd, gt_smem[4 * g_pad + j], blab)
            return (biou, bcid, blab)

        shape = x0.shape
        init = (jnp.full(shape, -1.0, jnp.float32),
                jnp.zeros(shape, jnp.int32),
                jnp.full(shape, -1.0, jnp.float32))
        biou, bcid, blab = lax.fori_loop(0, g_pad, step, init)
        pos = biou > _POS_THR
        neg = biou < _NEG_THR
        inds_ref[...] = jnp.where(pos, bcid + 1, jnp.where(neg, 0, -1))
        mo_ref[...] = biou
        labs_ref[...] = jnp.where(pos, blab.astype(jnp.int32), -1)

    return pl.pallas_call(
        body,
        grid=(nbt,),
        in_specs=[
            pl.BlockSpec(memory_space=pltpu.SMEM),
            pl.BlockSpec((4, _TC_R, 128), lambda i: (0, i, 0)),
        ],
        out_specs=[
            pl.BlockSpec((_TC_R, 128), lambda i: (i, 0)),
            pl.BlockSpec((_TC_R, 128), lambda i: (i, 0)),
            pl.BlockSpec((_TC_R, 128), lambda i: (i, 0)),
        ],
        out_shape=[
            jax.ShapeDtypeStruct((nbt * _TC_R, 128), jnp.int32),
            jax.ShapeDtypeStruct((nbt * _TC_R, 128), jnp.float32),
            jax.ShapeDtypeStruct((nbt * _TC_R, 128), jnp.int32),
        ],
        scratch_shapes=[
            pltpu.SMEM((5, g_pad), jnp.float32),
            pltpu.SMEM((g_pad,), jnp.int32),
        ],
    )


_SC_FRAC = 0.42


def kernel(bboxes, targets, num_level_bboxes):
    n = bboxes.shape[0]
    g = targets.shape[0]
    g_pad = -(-g // _L) * _L

    tgt = jnp.pad(targets, ((0, g_pad - g), (0, 0)),
                  constant_values=-1.0)
    gt_flat = tgt.T.reshape(-1)
    anc_t = bboxes.T

    sc_quantum = _NW * _L * 4
    n_sc = int(n * _SC_FRAC) // sc_quantum * sc_quantum
    n_tc = n - n_sc

    outs = []
    if n_sc > 0:
        per_w = n_sc // _NW
        sc_call = _build_sc_call(n_sc, g_pad, per_w, k_unroll=4, g_unroll=1)
        sc_out = sc_call(anc_t[:, :n_sc].reshape(-1), gt_flat)
        outs.append(sc_out)
    if n_tc > 0:
        nbt = -(-n_tc // (_TC_R * 128))
        n_tc_pad = nbt * _TC_R * 128
        anc_tc = jnp.pad(anc_t[:, n_sc:], ((0, 0), (0, n_tc_pad - n_tc)))
        tc_call = _build_tc_call(nbt, g_pad)
        tc_out = tc_call(gt_flat, anc_tc.reshape(4, nbt * _TC_R, 128))
        outs.append(tuple(o.reshape(-1)[:n_tc] for o in tc_out))

    if len(outs) == 2:
        inds, mo, labs = (jnp.concatenate([a, b]) for a, b in zip(outs[0], outs[1]))
    else:
        inds, mo, labs = outs[0]
    return (inds.astype(jnp.int64),
            mo,
            labs.astype(jnp.int64))
